# baseline (speedup 1.0000x reference)
import functools

import jax
import jax.numpy as jnp
from jax import lax
from jax.experimental import pallas as pl
from jax.experimental.pallas import tpu as pltpu
from jax.experimental.pallas import tpu_sc as plsc

_NUM_CORES = 2
_NUM_SUBCORES = 16
_NUM_WORKERS = _NUM_CORES * _NUM_SUBCORES
_LANES = 16
_CHUNK = 16
_SC_BATCH = 2


def _sc_body(batch, maxlen, embed, x_hbm, pos_hbm, out_hbm,
             in0, in1, ou0, ou1, po0, po1,
             si0, si1, so0, so1, sp0, sp1):
    t_per_w = maxlen // _NUM_WORKERS
    n_k = t_per_w // _CHUNK
    n_kk = n_k // 2

    ins, outs, poss = (in0, in1), (ou0, ou1), (po0, po1)
    isems, osems, psems = (si0, si1), (so0, so1), (sp0, sp1)

    wid = lax.axis_index("s") * _NUM_CORES + lax.axis_index("c")
    t0 = wid * t_per_w

    def pos_cp(k, q):
        src = pos_hbm.at[pl.ds(t0 + k * _CHUNK, _CHUNK), :]
        return pltpu.make_async_copy(src, poss[q], psems[q])

    def in_cp(k, b, q):
        src = x_hbm.at[b, pl.ds(t0 + k * _CHUNK, _CHUNK), :]
        return pltpu.make_async_copy(src, ins[q], isems[q])

    def out_cp(k, b, q):
        dst = out_hbm.at[b, pl.ds(t0 + k * _CHUNK, _CHUNK), :]
        return pltpu.make_async_copy(outs[q], dst, osems[q])

    pos_cp(0, 0).start()
    in_cp(0, 0, 0).start()
    in_cp(0, 1 % batch, 1).start()

    def kk_body(kk, _):
        for kpar in range(2):
            k = kk * 2 + kpar
            for b in range(batch):
                q = b % 2
                if b == 0:
                    pos_cp(k, kpar).wait()
                    if kpar == 0:
                        pos_cp(k + 1, 1).start()
                    else:
                        @pl.when(kk < n_kk - 1)
                        def _():
                            pos_cp(k + 1, 0).start()
                in_cp(k, b, q).wait()
                b_prev = (b - 2) % batch
                if b >= 2:
                    out_cp(k, b_prev, q).wait()
                else:
                    @pl.when(k > 0)
                    def _():
                        out_cp(k - 1, b_prev, q).wait()

                @plsc.parallel_loop(0, _CHUNK * 8, 1, unroll=4)
                def add_body(g):
                    r = g // 8
                    cb = (g % 8) * 128
                    for j in range(8):
                        c = cb + j * _LANES
                        outs[q][r, pl.ds(c, _LANES)] = (
                            ins[q][r, pl.ds(c, _LANES)]
                            + poss[kpar][r, pl.ds(c, _LANES)]
                        )

                out_cp(k, b, q).start()
                b_next = (b + 2) % batch
                if b + 2 < batch:
                    in_cp(k, b_next, q).start()
                elif kpar == 0:
                    in_cp(k + 1, b_next, q).start()
                else:
                    @pl.when(kk < n_kk - 1)
                    def _():
                        in_cp(k + 1, b_next, q).start()
        return 0

    lax.fori_loop(0, n_kk, kk_body, 0)
    out_cp(n_k - 1, (batch - 2) % batch, 0).wait()
    out_cp(n_k - 1, batch - 1, 1).wait()


def _sc_call(x, pos_table):
    batch, maxlen, embed = x.shape
    mesh = plsc.VectorSubcoreMesh(core_axis_name="c", subcore_axis_name="s")
    fn = functools.partial(_sc_body, batch, maxlen, embed)
    buf = pltpu.VMEM((_CHUNK, embed), jnp.float32)
    return pl.kernel(
        fn,
        mesh=mesh,
        out_type=jax.ShapeDtypeStruct((batch, maxlen, embed), x.dtype),
        scratch_types=[buf] * 6 + [pltpu.SemaphoreType.DMA] * 6,
        compiler_params=pltpu.CompilerParams(use_tc_tiling_on_sc=True),
    )(x, pos_table)


def _tc_add_body(x_ref, pos_ref, o_ref):
    o_ref[...] = x_ref[...] + pos_ref[...]


def _tc_call(x, pos_table):
    batch, maxlen, embed = x.shape
    t_chunk = 512
    grid = (maxlen // t_chunk, batch)
    return pl.pallas_call(
        _tc_add_body,
        grid=grid,
        in_specs=[
            pl.BlockSpec((1, t_chunk, embed), lambda t, b: (b, t, 0)),
            pl.BlockSpec((t_chunk, embed), lambda t, b: (t, 0)),
        ],
        out_specs=pl.BlockSpec((1, t_chunk, embed), lambda t, b: (b, t, 0)),
        out_shape=jax.ShapeDtypeStruct((batch, maxlen, embed), x.dtype),
    )(x, pos_table)


def kernel(x, pos_table):
    sc_out = _sc_call(x[:_SC_BATCH], pos_table)
    tc_out = _tc_call(x[_SC_BATCH:], pos_table)
    return jnp.concatenate([sc_out, tc_out], axis=0)


# speedup vs baseline: 2.2020x; 2.2020x over previous
import functools

import jax
import jax.numpy as jnp
from jax import lax
from jax.experimental import pallas as pl
from jax.experimental.pallas import tpu as pltpu
from jax.experimental.pallas import tpu_sc as plsc

_NUM_CORES = 2
_NUM_SUBCORES = 16
_NUM_WORKERS = _NUM_CORES * _NUM_SUBCORES
_LANES = 16
_CHUNK = 16


def _sc_body(batch, maxlen, embed, x_hbm, pos_hbm, out_hbm,
             in0, in1, ou0, ou1, po0, po1,
             si0, si1, so0, so1, sp0, sp1):
    t_per_w = maxlen // _NUM_WORKERS
    n_k = t_per_w // _CHUNK
    n_kk = n_k // 2

    ins, outs, poss = (in0, in1), (ou0, ou1), (po0, po1)
    isems, osems, psems = (si0, si1), (so0, so1), (sp0, sp1)

    wid = lax.axis_index("s") * _NUM_CORES + lax.axis_index("c")
    t0 = wid * t_per_w

    def pos_cp(k, q):
        src = pos_hbm.at[pl.ds(t0 + k * _CHUNK, _CHUNK), :]
        return pltpu.make_async_copy(src, poss[q], psems[q])

    def in_cp(k, b, q):
        src = x_hbm.at[b, pl.ds(t0 + k * _CHUNK, _CHUNK), :]
        return pltpu.make_async_copy(src, ins[q], isems[q])

    def out_cp(k, b, q):
        dst = out_hbm.at[b, pl.ds(t0 + k * _CHUNK, _CHUNK), :]
        return pltpu.make_async_copy(outs[q], dst, osems[q])

    pos_cp(0, 0).start()
    in_cp(0, 0, 0).start()
    in_cp(0, 1 % batch, 1).start()

    def kk_body(kk, _):
        for kpar in range(2):
            k = kk * 2 + kpar
            for b in range(batch):
                q = b % 2
                if b == 0:
                    pos_cp(k, kpar).wait()
                    if kpar == 0:
                        pos_cp(k + 1, 1).start()
                    else:
                        @pl.when(kk < n_kk - 1)
                        def _():
                            pos_cp(k + 1, 0).start()
                in_cp(k, b, q).wait()
                b_prev = (b - 2) % batch
                if b >= 2:
                    out_cp(k, b_prev, q).wait()
                else:
                    @pl.when(k > 0)
                    def _():
                        out_cp(k - 1, b_prev, q).wait()

                @plsc.parallel_loop(0, _CHUNK * 8, 1, unroll=4)
                def add_body(g):
                    r = g // 8
                    cb = (g % 8) * 128
                    for j in range(8):
                        c = cb + j * _LANES
                        outs[q][r, pl.ds(c, _LANES)] = (
                            ins[q][r, pl.ds(c, _LANES)]
                            + poss[kpar][r, pl.ds(c, _LANES)]
                        )

                out_cp(k, b, q).start()
                b_next = (b + 2) % batch
                if b + 2 < batch:
                    in_cp(k, b_next, q).start()
                elif kpar == 0:
                    in_cp(k + 1, b_next, q).start()
                else:
                    @pl.when(kk < n_kk - 1)
                    def _():
                        in_cp(k + 1, b_next, q).start()
        return 0

    lax.fori_loop(0, n_kk, kk_body, 0)
    out_cp(n_k - 1, (batch - 2) % batch, 0).wait()
    out_cp(n_k - 1, batch - 1, 1).wait()


def kernel(x, pos_table):
    batch, maxlen, embed = x.shape
    mesh = plsc.VectorSubcoreMesh(core_axis_name="c", subcore_axis_name="s")
    fn = functools.partial(_sc_body, batch, maxlen, embed)
    buf = pltpu.VMEM((_CHUNK, embed), jnp.float32)
    return pl.kernel(
        fn,
        mesh=mesh,
        out_type=jax.ShapeDtypeStruct((batch, maxlen, embed), x.dtype),
        scratch_types=[buf] * 6 + [pltpu.SemaphoreType.DMA] * 6,
        compiler_params=pltpu.CompilerParams(use_tc_tiling_on_sc=True),
    )(x, pos_table)


# speedup vs baseline: 2.2046x; 1.0012x over previous
import functools

import jax
import jax.numpy as jnp
from jax import lax
from jax.experimental import pallas as pl
from jax.experimental.pallas import tpu as pltpu
from jax.experimental.pallas import tpu_sc as plsc

_NUM_CORES = 2
_NUM_SUBCORES = 16
_NUM_WORKERS = _NUM_CORES * _NUM_SUBCORES
_LANES = 16
_CHUNK = 16


def _sc_body(batch, maxlen, embed, x_hbm, pos_hbm, out_hbm,
             in0, in1, ou0, ou1, po0, po1,
             si0, si1, so0, so1, sp0, sp1):
    t_per_w = maxlen // _NUM_WORKERS
    n_k = t_per_w // _CHUNK
    n_kk = n_k // 2

    ins, outs, poss = (in0, in1), (ou0, ou1), (po0, po1)
    isems, osems, psems = (si0, si1), (so0, so1), (sp0, sp1)

    wid = lax.axis_index("s") * _NUM_CORES + lax.axis_index("c")
    t0 = wid * t_per_w

    def pos_cp(k, q):
        src = pos_hbm.at[pl.ds(t0 + k * _CHUNK, _CHUNK), :]
        return pltpu.make_async_copy(src, poss[q], psems[q])

    def in_cp(k, b, q):
        src = x_hbm.at[b, pl.ds(t0 + k * _CHUNK, _CHUNK), :]
        return pltpu.make_async_copy(src, ins[q], isems[q])

    def out_cp(k, b, q):
        dst = out_hbm.at[b, pl.ds(t0 + k * _CHUNK, _CHUNK), :]
        return pltpu.make_async_copy(outs[q], dst, osems[q])

    pos_cp(0, 0).start()
    in_cp(0, 0, 0).start()
    in_cp(0, 1 % batch, 1).start()

    def kk_body(kk, _):
        for kpar in range(2):
            k = kk * 2 + kpar
            for b in range(batch):
                q = b % 2
                if b == 0:
                    pos_cp(k, kpar).wait()
                    if kpar == 0:
                        pos_cp(k + 1, 1).start()
                    else:
                        @pl.when(kk < n_kk - 1)
                        def _():
                            pos_cp(k + 1, 0).start()
                in_cp(k, b, q).wait()
                b_prev = (b - 2) % batch
                if b >= 2:
                    out_cp(k, b_prev, q).wait()
                else:
                    @pl.when(k > 0)
                    def _():
                        out_cp(k - 1, b_prev, q).wait()

                @plsc.parallel_loop(0, _CHUNK * 8, 1, unroll=4)
                def add_body(g):
                    r = g // 8
                    cb = (g % 8) * 128
                    for j in range(8):
                        c = cb + j * _LANES
                        outs[q][r, pl.ds(c, _LANES)] = (
                            ins[q][r, pl.ds(c, _LANES)]
                            + poss[kpar][r, pl.ds(c, _LANES)]
                        )

                out_cp(k, b, q).start()
                b_next = (b + 2) % batch
                if b + 2 < batch:
                    in_cp(k, b_next, q).start()
                elif kpar == 0:
                    in_cp(k + 1, b_next, q).start()
                else:
                    @pl.when(kk < n_kk - 1)
                    def _():
                        in_cp(k + 1, b_next, q).start()
        return 0

    lax.fori_loop(0, n_kk, kk_body, 0)
    out_cp(n_k - 1, (batch - 2) % batch, 0).wait()
    out_cp(n_k - 1, batch - 1, 1).wait()


def kernel(x, pos_table):
    batch, maxlen, embed = x.shape
    mesh = plsc.VectorSubcoreMesh(core_axis_name="c", subcore_axis_name="s")
    fn = functools.partial(_sc_body, batch, maxlen, embed)
    buf = pltpu.VMEM((_CHUNK, embed), jnp.float32)
    return pl.kernel(
        fn,
        mesh=mesh,
        out_type=jax.ShapeDtypeStruct((batch, maxlen, embed), x.dtype),
        scratch_types=[buf] * 6 + [pltpu.SemaphoreType.DMA] * 6,
        compiler_params=pltpu.CompilerParams(
            use_tc_tiling_on_sc=True,
            skip_device_barrier=True,
            disable_bounds_checks=True,
        ),
    )(x, pos_table)


# speedup vs baseline: 2.2080x; 1.0015x over previous
import functools

import jax
import jax.numpy as jnp
from jax import lax
from jax.experimental import pallas as pl
from jax.experimental.pallas import tpu as pltpu
from jax.experimental.pallas import tpu_sc as plsc

_NUM_CORES = 2
_NUM_SUBCORES = 16
_NUM_WORKERS = _NUM_CORES * _NUM_SUBCORES
_LANES = 16
_CHUNK = 16


def _sc_body(batch, maxlen, embed, x_hbm, pos_hbm, out_hbm,
             in0, in1, ou0, ou1, po0, po1,
             si0, si1, so0, so1, sp0, sp1):
    t_per_w = maxlen // _NUM_WORKERS
    n_k = t_per_w // _CHUNK
    n_kk = n_k // 2

    ins, outs, poss = (in0, in1), (ou0, ou1), (po0, po1)
    isems, osems, psems = (si0, si1), (so0, so1), (sp0, sp1)

    wid = lax.axis_index("s") * _NUM_CORES + lax.axis_index("c")
    t0 = wid * t_per_w

    def pos_cp(k, q):
        src = pos_hbm.at[pl.ds(t0 + k * _CHUNK, _CHUNK), :]
        return pltpu.make_async_copy(src, poss[q], psems[q])

    def in_cp(k, b, q):
        src = x_hbm.at[b, pl.ds(t0 + k * _CHUNK, _CHUNK), :]
        return pltpu.make_async_copy(src, ins[q], isems[q])

    def out_cp(k, b, q):
        dst = out_hbm.at[b, pl.ds(t0 + k * _CHUNK, _CHUNK), :]
        return pltpu.make_async_copy(outs[q], dst, osems[q])

    pos_cp(0, 0).start()
    in_cp(0, 0, 0).start()
    in_cp(0, 1 % batch, 1).start()

    def kk_body(kk, _):
        for kpar in range(2):
            k = kk * 2 + kpar
            for b in range(batch):
                q = b % 2
                if b == 0:
                    pos_cp(k, kpar).wait()
                    if kpar == 0:
                        pos_cp(k + 1, 1).start()
                    else:
                        @pl.when(kk < n_kk - 1)
                        def _():
                            pos_cp(k + 1, 0).start()
                in_cp(k, b, q).wait()
                b_prev = (b - 2) % batch
                if b >= 2:
                    out_cp(k, b_prev, q).wait()
                else:
                    @pl.when(k > 0)
                    def _():
                        out_cp(k - 1, b_prev, q).wait()

                @plsc.parallel_loop(0, _CHUNK * 8, 1, unroll=4)
                def add_body(g):
                    r = g // 8
                    cb = (g % 8) * 128
                    for j in range(8):
                        c = cb + j * _LANES
                        outs[q][r, pl.ds(c, _LANES)] = (
                            ins[q][r, pl.ds(c, _LANES)]
                            + poss[kpar][r, pl.ds(c, _LANES)]
                        )

                out_cp(k, b, q).start()
                b_next = (b + 2) % batch
                if b + 2 < batch:
                    in_cp(k, b_next, q).start()
                elif kpar == 0:
                    in_cp(k + 1, b_next, q).start()
                else:
                    @pl.when(kk < n_kk - 1)
                    def _():
                        in_cp(k + 1, b_next, q).start()
        return 0

    lax.fori_loop(0, n_kk, kk_body, 0)
    out_cp(n_k - 1, (batch - 2) % batch, 0).wait()
    out_cp(n_k - 1, batch - 1, 1).wait()


def kernel(x, pos_table):
    batch, maxlen, embed = x.shape
    mesh = plsc.VectorSubcoreMesh(core_axis_name="c", subcore_axis_name="s")
    fn = functools.partial(_sc_body, batch, maxlen, embed)
    buf = pltpu.VMEM((_CHUNK, embed), jnp.float32)
    return pl.kernel(
        fn,
        mesh=mesh,
        out_type=jax.ShapeDtypeStruct((batch, maxlen, embed), x.dtype),
        scratch_types=[buf] * 6 + [pltpu.SemaphoreType.DMA] * 6,
        compiler_params=pltpu.CompilerParams(use_tc_tiling_on_sc=True),
    )(x, pos_table)
